# Initial kernel scaffold; baseline (speedup 1.0000x reference)
import jax
import jax.numpy as jnp
from jax.experimental import pallas as pl


def kernel(fm2, fm3, fm4, fm5, rois, batch_indices):
    raise NotImplementedError("write your pallas kernel here")


# speedup vs baseline: 18.2018x; 18.2018x over previous
import jax
import jax.numpy as jnp
from jax import lax
from jax.experimental import pallas as pl
from jax.experimental.pallas import tpu as pltpu
from jax.experimental.pallas import tpu_sc as plsc

_C = 256
_NPAD = 1024
_NW = 32
_RPW = _NPAD // _NW
_PPW = _RPW * 49
_BATCH = 32
_NB = _PPW // _BATCH
_BASES = (0, 32768, 40960, 43008)
_SCALES = (0.248046875, 0.123046875, 0.060546875, 0.029296875)
_HS = (128, 64, 32, 16)
_T0, _T1, _T2 = 6272.0, 25088.0, 100352.0


def _compute_idx(bb, rois_v, bi_v, idx_ref, wx_ref, wy_ref):
    dyn0 = jnp.minimum(bi_v[pl.ds(0, 16)], 0)
    for g in range(2):
        lane = lax.iota(jnp.int32, 16)
        p = bb * _BATCH + g * 16 + lane + dyn0
        n = p // 49
        r49 = p - n * 49
        j = r49 // 7
        i = r49 - j * 7
        c0 = jnp.zeros((16,), jnp.int32)
        x1 = plsc.load_gather(rois_v, [n, c0])
        y1 = plsc.load_gather(rois_v, [n, c0 + 1])
        x2 = plsc.load_gather(rois_v, [n, c0 + 2])
        y2 = plsc.load_gather(rois_v, [n, c0 + 3])
        b = plsc.load_gather(bi_v, [n])
        hw = (x2 - x1) * (y2 - y1)
        lvl = ((hw >= _T0).astype(jnp.int32) + (hw >= _T1).astype(jnp.int32)
               + (hw >= _T2).astype(jnp.int32))
        scale = jnp.where(lvl == 0, _SCALES[0],
                          jnp.where(lvl == 1, _SCALES[1],
                                    jnp.where(lvl == 2, _SCALES[2], _SCALES[3])))
        hs = jnp.where(lvl == 0, _HS[0],
                       jnp.where(lvl == 1, _HS[1],
                                 jnp.where(lvl == 2, _HS[2], _HS[3])))
        bas = jnp.where(lvl == 0, _BASES[0],
                        jnp.where(lvl == 1, _BASES[1],
                                  jnp.where(lvl == 2, _BASES[2], _BASES[3])))
        in_y = (y1 + (j.astype(jnp.float32) * (1.0 / 6.0)) * (y2 - y1)) * scale
        in_x = (x1 + (i.astype(jnp.float32) * (1.0 / 6.0)) * (x2 - x1)) * scale
        y0 = in_y.astype(jnp.int32)
        x0 = in_x.astype(jnp.int32)
        wy = in_y - y0.astype(jnp.float32)
        wx = in_x - x0.astype(jnp.float32)
        hm1 = hs - 1
        y0c = jnp.minimum(y0, hm1)
        y1c = jnp.minimum(y0 + 1, hm1)
        x0c = jnp.minimum(x0, hm1)
        x1c = jnp.minimum(x0 + 1, hm1)
        common = bas + b * hs * hs
        row0 = common + y0c * hs
        row1 = common + y1c * hs
        idx_ref[pl.ds(g * 16, 16)] = row0 + x0c
        idx_ref[pl.ds(32 + g * 16, 16)] = row0 + x1c
        idx_ref[pl.ds(64 + g * 16, 16)] = row1 + x0c
        idx_ref[pl.ds(96 + g * 16, 16)] = row1 + x1c
        wx_ref[pl.ds(g * 16, 16)] = wx
        wy_ref[pl.ds(g * 16, 16)] = wy


def _combine(rows_ref, wx_ref, wy_ref, out_ref):
    def body(p, carry):
        pv = jnp.broadcast_to(p, (16,)).astype(jnp.int32)
        wxv = plsc.load_gather(wx_ref, [pv])
        wyv = plsc.load_gather(wy_ref, [pv])
        for c in range(_C // 16):
            sl = pl.ds(c * 16, 16)
            tl = rows_ref[p, sl]
            tr = rows_ref[p + 32, sl]
            bl = rows_ref[p + 64, sl]
            br = rows_ref[p + 96, sl]
            top = tl + wxv * (tr - tl)
            bot = bl + wxv * (br - bl)
            out_ref[p, sl] = top + wyv * (bot - top)
        return carry
    lax.fori_loop(0, _BATCH, body, 0)


def _sc_body(table, rois_h, bi_h, out_h, rois_v, bi_v, idx_a, idx_b,
             wx_a, wy_a, wx_b, wy_b, rows_a, rows_b, out_v, sg):
    cid = lax.axis_index("c")
    sid = lax.axis_index("s")
    wid = sid * 2 + cid
    pltpu.sync_copy(rois_h.at[pl.ds(wid * _RPW, _RPW)], rois_v)
    pltpu.sync_copy(bi_h.at[pl.ds(wid * _RPW, _RPW)], bi_v)
    pix0 = wid * _PPW

    def loop_body(g2, carry):
        a = 2 * g2

        @pl.when(g2 == 0)
        def _():
            _compute_idx(a, rois_v, bi_v, idx_a, wx_a, wy_a)
            pltpu.make_async_copy(table.at[idx_a], rows_a, sg).start()

        pltpu.make_async_copy(table.at[idx_a], rows_a, sg).wait()
        _compute_idx(a + 1, rois_v, bi_v, idx_b, wx_b, wy_b)
        pltpu.make_async_copy(table.at[idx_b], rows_b, sg).start()
        _combine(rows_a, wx_a, wy_a, out_v)
        pltpu.sync_copy(out_v, out_h.at[pl.ds(pix0 + a * _BATCH, _BATCH)])

        pltpu.make_async_copy(table.at[idx_b], rows_b, sg).wait()

        @pl.when(a + 2 < _NB)
        def _():
            _compute_idx(a + 2, rois_v, bi_v, idx_a, wx_a, wy_a)
            pltpu.make_async_copy(table.at[idx_a], rows_a, sg).start()
        _combine(rows_b, wx_b, wy_b, out_v)
        pltpu.sync_copy(out_v, out_h.at[pl.ds(pix0 + (a + 1) * _BATCH, _BATCH)])
        return carry

    lax.fori_loop(0, (_NB - 1) // 2, loop_body, 0)
    last = _NB - 1
    pltpu.make_async_copy(table.at[idx_a], rows_a, sg).wait()
    _combine(rows_a, wx_a, wy_a, out_v)
    pltpu.sync_copy(out_v, out_h.at[pl.ds(pix0 + last * _BATCH, _BATCH)])


def kernel(fm2, fm3, fm4, fm5, rois, batch_indices):
    n = rois.shape[0]
    c = fm2.shape[3]
    table = jnp.concatenate(
        [fm2.reshape(-1, c), fm3.reshape(-1, c), fm4.reshape(-1, c),
         fm5.reshape(-1, c)], axis=0)
    rois_p = jnp.pad(rois, ((0, _NPAD - n), (0, 0)))
    bi_p = jnp.pad(batch_indices.astype(jnp.int32), (0, _NPAD - n))

    mesh = plsc.VectorSubcoreMesh(core_axis_name="c", subcore_axis_name="s")
    run = pl.kernel(
        _sc_body,
        out_type=jax.ShapeDtypeStruct((_NPAD * 49, _C), jnp.float32),
        mesh=mesh,
        compiler_params=pltpu.CompilerParams(needs_layout_passes=False),
        scratch_types=[
            pltpu.VMEM((_RPW, 4), jnp.float32),
            pltpu.VMEM((_RPW,), jnp.int32),
            pltpu.VMEM((4 * _BATCH,), jnp.int32),
            pltpu.VMEM((4 * _BATCH,), jnp.int32),
            pltpu.VMEM((_BATCH,), jnp.float32),
            pltpu.VMEM((_BATCH,), jnp.float32),
            pltpu.VMEM((_BATCH,), jnp.float32),
            pltpu.VMEM((_BATCH,), jnp.float32),
            pltpu.VMEM((4 * _BATCH, _C), jnp.float32),
            pltpu.VMEM((4 * _BATCH, _C), jnp.float32),
            pltpu.VMEM((_BATCH, _C), jnp.float32),
            pltpu.SemaphoreType.DMA,
        ],
    )
    out_flat = run(table, rois_p, bi_p)
    roi_features = out_flat[: n * 49].reshape(n, 7, 7, c)
    return (roi_features, batch_indices.astype(jnp.int32))
